# baseline (speedup 1.0000x reference)
import jax
import jax.numpy as jnp
from jax import lax
from jax.experimental import pallas as pl
from jax.experimental.pallas import tpu as pltpu

WIN = 15
WLEN = 2 * WIN + 1
WPAD = 40
NSTR = 4


def _body(*refs):
    gap_ref, bgap_ref, bcls_ref = refs[0:3]
    x_refs = refs[3:3 + NSTR]
    bm_ref, pooled_ref, wg_ref, wc_ref = refs[3 + NSTR:7 + NSTR]
    out_ref = refs[7 + NSTR]

    b = pl.program_id(0)
    S = x_refs[0].shape[1]
    HC = x_refs[0].shape[2]
    H = HC * NSTR
    G = gap_ref.shape[1]

    bm = bm_ref[0, :, :]
    tcnt = jnp.sum(bm)

    def wchunk(ref, part, k):
        return ref[0:1, part * H + k * HC: part * H + (k + 1) * HC]

    cls_score = bcls_ref[0] + jnp.zeros((1, 1), jnp.float32)
    tmaxs, tavgs = [], []
    for k in range(NSTR):
        xb = x_refs[k][0] * bm
        tmax = jnp.max(xb, axis=0, keepdims=True)
        tsum = jnp.sum(xb, axis=0, keepdims=True)
        tavg = tsum / tcnt
        pooled = pooled_ref[0, 0:1, k * HC:(k + 1) * HC]
        cls_score = (cls_score
                     + jnp.sum(pooled * wchunk(wc_ref, 0, k), axis=1, keepdims=True)
                     + jnp.sum(tmax * wchunk(wc_ref, 1, k), axis=1, keepdims=True)
                     + jnp.sum(tavg * wchunk(wc_ref, 2, k), axis=1, keepdims=True))

    scores = [cls_score]
    for g in range(G):
        gid = gap_ref[b, g]
        lo = gid - WIN
        hi = gid + WIN
        d = jnp.clip(lo, 0, S - WPAD)
        d = pl.multiple_of(jnp.minimum((d // 8) * 8, S - WPAD), 8)
        bmr = bm_ref[0, pl.ds(d, WPAD), :]
        pos = d + lax.broadcasted_iota(jnp.int32, (WPAD, 1), 0)
        inwin = jnp.logical_and(pos >= lo, pos <= hi).astype(jnp.float32)
        rowm = inwin * bmr
        cnt = jnp.sum(rowm)
        gm = (pos == gid).astype(jnp.float32)
        sc = bgap_ref[0] + jnp.zeros((1, 1), jnp.float32)
        for k in range(NSTR):
            sl = x_refs[k][0, pl.ds(d, WPAD), :]
            m = sl * rowm
            wmax = jnp.maximum(jnp.max(m, axis=0, keepdims=True), 0.0)
            wsum = jnp.sum(m, axis=0, keepdims=True)
            gaprow = jnp.sum(sl * gm, axis=0, keepdims=True)
            sc = (sc
                  + jnp.sum(gaprow * wchunk(wg_ref, 0, k), axis=1, keepdims=True)
                  + jnp.sum(wmax * wchunk(wg_ref, 1, k), axis=1, keepdims=True)
                  + jnp.sum((wsum / cnt) * wchunk(wg_ref, 2, k), axis=1, keepdims=True))
        scores.append(sc)

    out_ref[0] = jnp.concatenate(scores, axis=0)


def kernel(sequence_output, pooled_output, token_type_ids, word_mask, gap_ids,
           W_gap, b_gap, W_cls, b_cls):
    B, S, H = sequence_output.shape
    G = gap_ids.shape[1]
    HC = H // NSTR
    bm = ((token_type_ids == 0).astype(jnp.int32) * word_mask
          ).astype(jnp.float32)[..., None]
    pooled3 = pooled_output[:, None, :]
    x_specs = [
        pl.BlockSpec((1, S, HC), lambda b, k=k: (b, 0, k))
        for k in range(NSTR)
    ]
    out = pl.pallas_call(
        _body,
        grid=(B,),
        in_specs=[
            pl.BlockSpec(memory_space=pltpu.SMEM),
            pl.BlockSpec(memory_space=pltpu.SMEM),
            pl.BlockSpec(memory_space=pltpu.SMEM),
            *x_specs,
            pl.BlockSpec((1, S, 1), lambda b: (b, 0, 0)),
            pl.BlockSpec((1, 1, H), lambda b: (b, 0, 0)),
            pl.BlockSpec((1, 3 * H), lambda b: (0, 0)),
            pl.BlockSpec((1, 3 * H), lambda b: (0, 0)),
        ],
        out_specs=pl.BlockSpec((1, 1 + G, 1), lambda b: (b, 0, 0)),
        out_shape=jax.ShapeDtypeStruct((B, 1 + G, 1), jnp.float32),
    )(gap_ids, b_gap, b_cls,
      *([sequence_output] * NSTR),
      bm, pooled3, W_gap, W_cls)
    return out[:, :, 0]
